# baseline (speedup 1.0000x reference)
import functools

import jax
import jax.numpy as jnp
from jax import lax
from jax.experimental import pallas as pl
from jax.experimental.pallas import tpu as pltpu
from jax.experimental.pallas import tpu_sc as plsc

NW = 32
NT = 16
CH = 128


def _zero_rows(buf, n_rows, n_colgrp):
    def zb(i, _):
        r = i // n_colgrp
        c = i % n_colgrp
        buf[r, pl.ds(c * 16, 16)] = jnp.zeros((16,), jnp.float32)
        return 0
    lax.fori_loop(0, n_rows * n_colgrp, zb, 0)


def _deg_call(dstp, npad, cpw):
    mesh = plsc.VectorSubcoreMesh(core_axis_name="c", subcore_axis_name="s")
    rpt = npad // NT
    nz = rpt // CH

    @functools.partial(
        pl.kernel, mesh=mesh,
        out_type=jax.ShapeDtypeStruct((2, npad, 16), jnp.float32),
        scratch_types=[
            pltpu.VMEM((cpw, CH), jnp.int32),
            pltpu.VMEM((CH, 16), jnp.float32),
            pltpu.VMEM_SHARED((npad, 16), jnp.float32),
        ],
    )
    def deg_k(dstp_hbm, out_hbm, dst_v, buf_v, acc):
        cid = lax.axis_index("c")
        sid = lax.axis_index("s")
        wid = cid * NT + sid
        _zero_rows(buf_v, CH, 1)

        def zs(k, _):
            pltpu.sync_copy(buf_v, acc.at[pl.ds(sid * rpt + k * CH, CH)])
            return 0
        lax.fori_loop(0, nz, zs, 0)

        def ob(i, _):
            buf_v[i, :] = jnp.ones((16,), jnp.float32)
            return 0
        lax.fori_loop(0, CH, ob, 0)
        plsc.subcore_barrier()

        pltpu.sync_copy(dstp_hbm.at[wid], dst_v)

        def body(j, _):
            pltpu.sync_copy(buf_v, acc.at[dst_v.at[j]], add=True)
            return 0
        lax.fori_loop(0, cpw, body, 0)
        plsc.subcore_barrier()

        pltpu.sync_copy(acc.at[pl.ds(sid * rpt, rpt)],
                        out_hbm.at[cid].at[pl.ds(sid * rpt, rpt)])

    return deg_k(dstp)


def _scatter_call(hp, srcp, dstp, npad, cpw):
    mesh = plsc.VectorSubcoreMesh(core_axis_name="c", subcore_axis_name="s")
    rpt = npad // NT
    nz = rpt // CH
    f = hp.shape[1]

    @functools.partial(
        pl.kernel, mesh=mesh,
        out_type=jax.ShapeDtypeStruct((2, npad, f), jnp.float32),
        scratch_types=[
            pltpu.VMEM((cpw, CH), jnp.int32),
            pltpu.VMEM((cpw, CH), jnp.int32),
            pltpu.VMEM((CH, f), jnp.float32),
            pltpu.VMEM_SHARED((npad, f), jnp.float32),
            pltpu.SemaphoreType.DMA,
        ],
    )
    def scat_k(hp_hbm, srcp_hbm, dstp_hbm, out_hbm, src_v, dst_v, rows_v,
               acc, sem):
        cid = lax.axis_index("c")
        sid = lax.axis_index("s")
        wid = cid * NT + sid
        _zero_rows(rows_v, CH, f // 16)

        def zs(k, _):
            pltpu.sync_copy(rows_v, acc.at[pl.ds(sid * rpt + k * CH, CH)])
            return 0
        lax.fori_loop(0, nz, zs, 0)

        pltpu.sync_copy(srcp_hbm.at[wid], src_v)
        pltpu.sync_copy(dstp_hbm.at[wid], dst_v)
        plsc.subcore_barrier()

        def body(j, _):
            pltpu.async_copy(hp_hbm.at[src_v.at[j]], rows_v, sem).wait()
            pltpu.sync_copy(rows_v, acc.at[dst_v.at[j]], add=True)
            return 0
        lax.fori_loop(0, cpw, body, 0)
        plsc.subcore_barrier()

        pltpu.sync_copy(acc.at[pl.ds(sid * rpt, rpt)],
                        out_hbm.at[cid].at[pl.ds(sid * rpt, rpt)])

    return scat_k(hp, srcp, dstp)


def _tc1_body(x_ref, w_ref, degp_ref, o_ref):
    h = jnp.dot(x_ref[...], w_ref[...], preferred_element_type=jnp.float32)
    deg = degp_ref[0, :, 0:1] + degp_ref[1, :, 0:1] + 1.0
    o_ref[...] = h * lax.rsqrt(deg)


def _tc2_body(sp_ref, hp_ref, degp_ref, b_ref, w_ref, o_ref):
    s = sp_ref[0] + sp_ref[1]
    deg = degp_ref[0, :, 0:1] + degp_ref[1, :, 0:1] + 1.0
    dinv = lax.rsqrt(deg)
    o1 = jnp.tanh(dinv * (s + hp_ref[...]) + b_ref[...])
    h2 = jnp.dot(o1, w_ref[...], preferred_element_type=jnp.float32)
    o_ref[...] = h2 * dinv


def _tc3_body(sp_ref, hp_ref, degp_ref, b_ref, wc_ref, bc_ref,
              logit_ref, h_ref):
    s = sp_ref[0] + sp_ref[1]
    deg = degp_ref[0, :, 0:1] + degp_ref[1, :, 0:1] + 1.0
    dinv = lax.rsqrt(deg)
    o2 = jnp.tanh(dinv * (s + hp_ref[...]) + b_ref[...])
    h_ref[...] = o2
    logit_ref[...] = (
        jnp.dot(o2, wc_ref[...], preferred_element_type=jnp.float32)
        + bc_ref[...])


def kernel(x, edge_index, W1, b1, W2, b2, Wc, bc):
    n, f = x.shape
    e = edge_index.shape[1]
    n_cls = Wc.shape[1]
    cpw = max(88, 4 * (-(-e // (NW * CH * 4))))
    ep = NW * cpw * CH
    npad = -(-(n + 1) // (NT * CH)) * NT * CH

    src = edge_index[0]
    dst = edge_index[1]
    pad = ep - e
    srcp = jnp.concatenate(
        [src, jnp.zeros((pad,), jnp.int32)]).reshape(NW, cpw, CH)
    dummy = n + jnp.arange(pad, dtype=jnp.int32) % jnp.int32(npad - n)
    dstp = jnp.concatenate([dst, dummy]).reshape(NW, cpw, CH)
    b1r = b1.reshape(1, -1)
    b2r = b2.reshape(1, -1)
    bcr = bc.reshape(1, -1)

    degp = _deg_call(dstp, npad, cpw)

    rb = 1000
    grid = (n // rb,)
    deg_spec = pl.BlockSpec((2, rb, 16), lambda i: (0, i, 0))
    row_spec = pl.BlockSpec((rb, f), lambda i: (i, 0))
    sp_spec = pl.BlockSpec((2, rb, f), lambda i: (0, i, 0))
    w_spec = pl.BlockSpec((f, f), lambda i: (0, 0))
    b_spec = pl.BlockSpec((1, f), lambda i: (0, 0))

    hp1 = pl.pallas_call(
        _tc1_body,
        grid=grid,
        in_specs=[row_spec, w_spec, deg_spec],
        out_specs=row_spec,
        out_shape=jax.ShapeDtypeStruct((n, f), jnp.float32),
    )(x, W1, degp)

    s1 = _scatter_call(hp1, srcp, dstp, npad, cpw)

    hp2 = pl.pallas_call(
        _tc2_body,
        grid=grid,
        in_specs=[sp_spec, row_spec, deg_spec, b_spec, w_spec],
        out_specs=row_spec,
        out_shape=jax.ShapeDtypeStruct((n, f), jnp.float32),
    )(s1, hp1, degp, b1r, W2)

    s2 = _scatter_call(hp2, srcp, dstp, npad, cpw)

    logits, h_out = pl.pallas_call(
        _tc3_body,
        grid=grid,
        in_specs=[sp_spec, row_spec, deg_spec, b_spec,
                  pl.BlockSpec((f, n_cls), lambda i: (0, 0)),
                  pl.BlockSpec((1, n_cls), lambda i: (0, 0))],
        out_specs=[pl.BlockSpec((rb, n_cls), lambda i: (i, 0)), row_spec],
        out_shape=[jax.ShapeDtypeStruct((n, n_cls), jnp.float32),
                   jax.ShapeDtypeStruct((n, f), jnp.float32)],
    )(s2, hp2, degp, b2r, Wc, bcr)

    return (logits, h_out)


# speedup vs baseline: 7.4630x; 7.4630x over previous
import functools

import jax
import jax.numpy as jnp
from jax import lax
from jax.experimental import pallas as pl
from jax.experimental.pallas import tpu as pltpu
from jax.experimental.pallas import tpu_sc as plsc

NW = 32
NT = 16
CH = 128


def _zero_rows(buf, n_rows, n_colgrp):
    def zb(i, _):
        r = i // n_colgrp
        c = i % n_colgrp
        buf[r, pl.ds(c * 16, 16)] = jnp.zeros((16,), jnp.float32)
        return 0
    lax.fori_loop(0, n_rows * n_colgrp, zb, 0)


def _deg_call(dstp, npad, cpw):
    mesh = plsc.VectorSubcoreMesh(core_axis_name="c", subcore_axis_name="s")
    rpt = npad // NT
    nz = rpt // CH

    @functools.partial(
        pl.kernel, mesh=mesh,
        out_type=jax.ShapeDtypeStruct((2, npad, 16), jnp.float32),
        scratch_types=[
            pltpu.VMEM((cpw, CH), jnp.int32),
            pltpu.VMEM((CH, 16), jnp.float32),
            pltpu.VMEM_SHARED((npad, 16), jnp.float32),
        ],
    )
    def deg_k(dstp_hbm, out_hbm, dst_v, buf_v, acc):
        cid = lax.axis_index("c")
        sid = lax.axis_index("s")
        wid = cid * NT + sid
        _zero_rows(buf_v, CH, 1)

        def zs(k, _):
            pltpu.sync_copy(buf_v, acc.at[pl.ds(sid * rpt + k * CH, CH)])
            return 0
        lax.fori_loop(0, nz, zs, 0)

        def ob(i, _):
            buf_v[i, :] = jnp.ones((16,), jnp.float32)
            return 0
        lax.fori_loop(0, CH, ob, 0)
        plsc.subcore_barrier()

        pltpu.sync_copy(dstp_hbm.at[wid], dst_v)

        def body(j, _):
            pltpu.sync_copy(buf_v, acc.at[dst_v.at[j]], add=True)
            return 0
        lax.fori_loop(0, cpw, body, 0)
        plsc.subcore_barrier()

        pltpu.sync_copy(acc.at[pl.ds(sid * rpt, rpt)],
                        out_hbm.at[cid].at[pl.ds(sid * rpt, rpt)])

    return deg_k(dstp)


def _scatter_call(hp, srcp, dstp, npad, cpw):
    mesh = plsc.VectorSubcoreMesh(core_axis_name="c", subcore_axis_name="s")
    rpt = npad // NT
    nz = rpt // CH
    f = hp.shape[1]

    @functools.partial(
        pl.kernel, mesh=mesh,
        out_type=jax.ShapeDtypeStruct((2, npad, f), jnp.float32),
        scratch_types=[
            pltpu.VMEM((cpw, CH), jnp.int32),
            pltpu.VMEM((cpw, CH), jnp.int32),
            pltpu.VMEM((CH, f), jnp.float32),
            pltpu.VMEM_SHARED((npad, f), jnp.float32),
            pltpu.SemaphoreType.DMA,
        ],
    )
    def scat_k(hp_hbm, srcp_hbm, dstp_hbm, out_hbm, src_v, dst_v, rows_v,
               acc, sem):
        cid = lax.axis_index("c")
        sid = lax.axis_index("s")
        wid = cid * NT + sid
        _zero_rows(rows_v, CH, f // 16)

        def zs(k, _):
            pltpu.sync_copy(rows_v, acc.at[pl.ds(sid * rpt + k * CH, CH)])
            return 0
        lax.fori_loop(0, nz, zs, 0)

        pltpu.sync_copy(srcp_hbm.at[wid], src_v)
        pltpu.sync_copy(dstp_hbm.at[wid], dst_v)
        plsc.subcore_barrier()

        def body(j, _):
            pltpu.async_copy(hp_hbm.at[src_v.at[j]], rows_v, sem).wait()
            pltpu.sync_copy(rows_v, acc.at[dst_v.at[j]], add=True)
            return 0
        lax.fori_loop(0, cpw, body, 0)
        plsc.subcore_barrier()

        pltpu.sync_copy(acc.at[pl.ds(sid * rpt, rpt)],
                        out_hbm.at[cid].at[pl.ds(sid * rpt, rpt)])

    return scat_k(hp, srcp, dstp)


def _tc1_body(x_ref, w_ref, degp_ref, o_ref):
    h = jnp.dot(x_ref[...], w_ref[...], preferred_element_type=jnp.float32)
    deg = degp_ref[0, :, 0:1] + degp_ref[1, :, 0:1] + 1.0
    o_ref[...] = h * lax.rsqrt(deg)


def _tc2_body(sp_ref, hp_ref, degp_ref, b_ref, w_ref, o_ref):
    s = sp_ref[0] + sp_ref[1]
    deg = degp_ref[0, :, 0:1] + degp_ref[1, :, 0:1] + 1.0
    dinv = lax.rsqrt(deg)
    o1 = jnp.tanh(dinv * (s + hp_ref[...]) + b_ref[...])
    h2 = jnp.dot(o1, w_ref[...], preferred_element_type=jnp.float32)
    o_ref[...] = h2 * dinv


def _tc3_body(sp_ref, hp_ref, degp_ref, b_ref, wc_ref, bc_ref,
              logit_ref, h_ref):
    s = sp_ref[0] + sp_ref[1]
    deg = degp_ref[0, :, 0:1] + degp_ref[1, :, 0:1] + 1.0
    dinv = lax.rsqrt(deg)
    o2 = jnp.tanh(dinv * (s + hp_ref[...]) + b_ref[...])
    h_ref[...] = o2
    logit_ref[...] = (
        jnp.dot(o2, wc_ref[...], preferred_element_type=jnp.float32)
        + bc_ref[...])


def kernel(x, edge_index, W1, b1, W2, b2, Wc, bc):
    n, f = x.shape
    e = edge_index.shape[1]
    n_cls = Wc.shape[1]
    cpw = 4 * (-(-e // (NW * CH * 4)))
    ep = NW * cpw * CH
    npad = -(-(n + 1) // (NT * CH)) * NT * CH

    src = edge_index[0]
    dst = edge_index[1]
    pad = ep - e
    ar = jnp.arange(pad, dtype=jnp.int32)
    srcp = jnp.concatenate(
        [src, ar % jnp.int32(n)]).reshape(NW, cpw, CH)
    dstp = jnp.concatenate(
        [dst, n + ar % jnp.int32(npad - n)]).reshape(NW, cpw, CH)
    b1r = b1.reshape(1, -1)
    b2r = b2.reshape(1, -1)
    bcr = bc.reshape(1, -1)

    degp = _deg_call(dstp, npad, cpw)

    rb = 1000
    grid = (n // rb,)
    deg_spec = pl.BlockSpec((2, rb, 16), lambda i: (0, i, 0))
    row_spec = pl.BlockSpec((rb, f), lambda i: (i, 0))
    sp_spec = pl.BlockSpec((2, rb, f), lambda i: (0, i, 0))
    w_spec = pl.BlockSpec((f, f), lambda i: (0, 0))
    b_spec = pl.BlockSpec((1, f), lambda i: (0, 0))

    hp1 = pl.pallas_call(
        _tc1_body,
        grid=grid,
        in_specs=[row_spec, w_spec, deg_spec],
        out_specs=row_spec,
        out_shape=jax.ShapeDtypeStruct((n, f), jnp.float32),
    )(x, W1, degp)

    s1 = _scatter_call(hp1, srcp, dstp, npad, cpw)

    hp2 = pl.pallas_call(
        _tc2_body,
        grid=grid,
        in_specs=[sp_spec, row_spec, deg_spec, b_spec, w_spec],
        out_specs=row_spec,
        out_shape=jax.ShapeDtypeStruct((n, f), jnp.float32),
    )(s1, hp1, degp, b1r, W2)

    s2 = _scatter_call(hp2, srcp, dstp, npad, cpw)

    logits, h_out = pl.pallas_call(
        _tc3_body,
        grid=grid,
        in_specs=[sp_spec, row_spec, deg_spec, b_spec,
                  pl.BlockSpec((f, n_cls), lambda i: (0, 0)),
                  pl.BlockSpec((1, n_cls), lambda i: (0, 0))],
        out_specs=[pl.BlockSpec((rb, n_cls), lambda i: (i, 0)), row_spec],
        out_shape=[jax.ShapeDtypeStruct((n, n_cls), jnp.float32),
                   jax.ShapeDtypeStruct((n, f), jnp.float32)],
    )(s2, hp2, degp, b2r, Wc, bcr)

    return (logits, h_out)
